# baseline (speedup 1.0000x reference)
import jax
import jax.numpy as jnp
from jax.experimental import pallas as pl


def _gcn(x, src, dst, W, b):
    N = x.shape[0]
    h = x @ W
    deg = jnp.zeros((N,), jnp.float32).at[dst].add(1.0)
    dinv = jnp.where(deg > 0, deg**-0.5, 0.0)
    norm = dinv[src] * dinv[dst]
    msgs = h[src] * norm[:, None]
    agg = jnp.zeros((N, h.shape[1]), jnp.float32).at[dst].add(msgs)
    return agg + b


def _ident(x_ref, o_ref):
    o_ref[...] = x_ref[...]


def kernel(x, edge_index, W1, b1, W2, b2):
    N = x.shape[0]
    loop = jnp.arange(N, dtype=edge_index.dtype)
    src = jnp.concatenate([edge_index[0], loop])
    dst = jnp.concatenate([edge_index[1], loop])
    h1 = jnp.tanh(_gcn(x, src, dst, W1, b1))
    h2 = jnp.tanh(_gcn(h1, src, dst, W2, b2))
    return pl.pallas_call(
        _ident, out_shape=jax.ShapeDtypeStruct(h2.shape, h2.dtype)
    )(h2)


# speedup vs baseline: 13.2467x; 13.2467x over previous
import functools

import jax
import jax.numpy as jnp
from jax import lax
from jax.experimental import pallas as pl
from jax.experimental.pallas import tpu as pltpu
from jax.experimental.pallas import tpu_sc as plsc

_N = 10000
_NPAD = 10240
_B = 128
_NBLK = 80
_NT = 32
_EPAD = _NT * _NBLK * _B
_SLABC = 5


def _make_edge_scatter(width, do_gather):
    mesh = plsc.VectorSubcoreMesh(core_axis_name="c", subcore_axis_name="s")

    def body(src_hbm, dst_hbm, tab_hbm, zeros_hbm, out_hbm,
             src_v, dst_v, rows_v, acc_sh, sem):
        cid = lax.axis_index("c")
        sid = lax.axis_index("s")
        t = cid * 16 + sid
        pltpu.sync_copy(dst_hbm.at[t], dst_v)
        if do_gather:
            pltpu.sync_copy(src_hbm.at[t], src_v)
        else:
            pltpu.sync_copy(tab_hbm, rows_v.at[0])
        for c in range(_SLABC):
            rows = pl.ds((sid * _SLABC + c) * _B, _B)
            pltpu.sync_copy(zeros_hbm.at[rows], rows_v.at[1])
            pltpu.sync_copy(rows_v.at[1], acc_sh.at[rows])
        plsc.subcore_barrier()

        if do_gather:
            def lbody(j, carry):
                pltpu.async_copy(tab_hbm.at[src_v.at[j]], rows_v.at[0], sem).wait()
                pltpu.sync_copy(rows_v.at[0], acc_sh.at[dst_v.at[j]], add=True)
                return carry
        else:
            def lbody(j, carry):
                pltpu.sync_copy(rows_v.at[0], acc_sh.at[dst_v.at[j]], add=True)
                return carry
        lax.fori_loop(0, _NBLK, lbody, 0)

        plsc.subcore_barrier()
        for c in range(_SLABC):
            rows = pl.ds((sid * _SLABC + c) * _B, _B)
            pltpu.sync_copy(acc_sh.at[rows], out_hbm.at[cid, rows])

    return pl.kernel(
        body,
        out_type=jax.ShapeDtypeStruct((2, _NPAD, width), jnp.float32),
        mesh=mesh,
        compiler_params=pltpu.CompilerParams(use_tc_tiling_on_sc=False),
        scratch_types=[
            pltpu.VMEM((_NBLK, _B), jnp.int32),
            pltpu.VMEM((_NBLK, _B), jnp.int32),
            pltpu.VMEM((2, _B, width), jnp.float32),
            pltpu.VMEM_SHARED((_NPAD, width), jnp.float32),
            pltpu.SemaphoreType.DMA,
        ],
    )


_deg_scatter = _make_edge_scatter(16, do_gather=False)
_g_scatter = _make_edge_scatter(64, do_gather=True)


def _mm_body(x_ref, w_ref, o_ref):
    o_ref[...] = jnp.dot(x_ref[...], w_ref[...], preferred_element_type=jnp.float32)


def _prep_body(degp_ref, h_ref, g_ref, dinv_ref):
    d = degp_ref[0] + degp_ref[1]
    deg = d[0:_N, 0:1] + 1.0
    dinv = lax.rsqrt(deg)
    dinv_ref[...] = dinv
    g_ref[...] = h_ref[...] * dinv


def _fin1_body(p_ref, g_ref, dinv_ref, b_ref, w_ref, o_ref):
    s = p_ref[0, 0:_N, :] + p_ref[1, 0:_N, :] + g_ref[...]
    h = jnp.tanh(s * dinv_ref[...] + b_ref[...])
    o_ref[...] = jnp.dot(h, w_ref[...], preferred_element_type=jnp.float32) * dinv_ref[...]


def _fin2_body(p_ref, g_ref, dinv_ref, b_ref, o_ref):
    s = p_ref[0, 0:_N, :] + p_ref[1, 0:_N, :] + g_ref[...]
    o_ref[...] = jnp.tanh(s * dinv_ref[...] + b_ref[...])


def _tc(body, *outs):
    return pl.pallas_call(body, out_shape=[jax.ShapeDtypeStruct(s, jnp.float32) for s in outs])


def kernel(x, edge_index, W1, b1, W2, b2):
    src = edge_index[0].astype(jnp.int32)
    dst = edge_index[1].astype(jnp.int32)
    e = src.shape[0]
    fill = jnp.full((_EPAD - e,), _N, jnp.int32)
    src_t = jnp.concatenate([src, fill]).reshape(_NT, _NBLK, _B)
    dst_t = jnp.concatenate([dst, fill]).reshape(_NT, _NBLK, _B)
    zeros16 = jnp.zeros((_NPAD, 16), jnp.float32)
    zeros64 = jnp.zeros((_NPAD, 64), jnp.float32)
    ones16 = jnp.ones((_B, 16), jnp.float32)

    degp = _deg_scatter(src_t, dst_t, ones16, zeros16)
    h1r, = _tc(_mm_body, (_N, 64))(x, W1)
    g1, dinv = _tc(_prep_body, (_N, 64), (_N, 1))(degp, h1r)
    g1p = jnp.pad(g1, ((0, _NPAD - _N), (0, 0)))
    p1 = _g_scatter(src_t, dst_t, g1p, zeros64)
    g2, = _tc(_fin1_body, (_N, 64))(p1, g1, dinv, b1.reshape(1, 64), W2)
    g2p = jnp.pad(g2, ((0, _NPAD - _N), (0, 0)))
    p2 = _g_scatter(src_t, dst_t, g2p, zeros64)
    out, = _tc(_fin2_body, (_N, 64))(p2, g2, dinv, b2.reshape(1, 64))
    return out


# speedup vs baseline: 41.1181x; 3.1040x over previous
import functools

import jax
import jax.numpy as jnp
from jax import lax
from jax.experimental import pallas as pl
from jax.experimental.pallas import tpu as pltpu
from jax.experimental.pallas import tpu_sc as plsc

_N = 10000
_NPAD = 10240
_B = 128
_NBLK = 80
_NT = 32
_EPAD = _NT * _NBLK * _B
_SLABC = 5
_NBUF = 4


def _make_edge_scatter(width, do_gather):
    mesh = plsc.VectorSubcoreMesh(core_axis_name="c", subcore_axis_name="s")

    def body(src_hbm, dst_hbm, tab_hbm, zeros_hbm, out_hbm,
             src_v, dst_v, rows_v, acc_sh, sem0, sem1, sem2, sem3):
        sems = (sem0, sem1, sem2, sem3)
        cid = lax.axis_index("c")
        sid = lax.axis_index("s")
        t = cid * 16 + sid
        pltpu.sync_copy(dst_hbm.at[t], dst_v)
        if do_gather:
            pltpu.sync_copy(src_hbm.at[t], src_v)
        else:
            pltpu.sync_copy(tab_hbm, rows_v.at[0])
        for c in range(_SLABC):
            rows = pl.ds((sid * _SLABC + c) * _B, _B)
            pltpu.sync_copy(zeros_hbm.at[rows], rows_v.at[1])
            pltpu.sync_copy(rows_v.at[1], acc_sh.at[rows])
        plsc.subcore_barrier()

        if do_gather:
            for b in range(_NBUF):
                pltpu.async_copy(tab_hbm.at[src_v.at[b]], rows_v.at[b], sems[b])

            def lbody(i, carry):
                j0 = i * _NBUF
                for b in range(_NBUF):
                    j = j0 + b
                    pltpu.make_async_copy(
                        tab_hbm.at[src_v.at[j]], rows_v.at[b], sems[b]).wait()
                    pltpu.sync_copy(rows_v.at[b], acc_sh.at[dst_v.at[j]], add=True)

                    @pl.when(j + _NBUF < _NBLK)
                    def _():
                        pltpu.async_copy(
                            tab_hbm.at[src_v.at[j + _NBUF]], rows_v.at[b], sems[b])
                return carry

            lax.fori_loop(0, _NBLK // _NBUF, lbody, 0)
        else:
            def lbody(j, carry):
                pltpu.sync_copy(rows_v.at[0], acc_sh.at[dst_v.at[j]], add=True)
                return carry

            lax.fori_loop(0, _NBLK, lbody, 0)

        plsc.subcore_barrier()
        for c in range(_SLABC):
            rows = pl.ds((sid * _SLABC + c) * _B, _B)
            pltpu.sync_copy(acc_sh.at[rows], out_hbm.at[cid, rows])

    return pl.kernel(
        body,
        out_type=jax.ShapeDtypeStruct((2, _NPAD, width), jnp.float32),
        mesh=mesh,
        compiler_params=pltpu.CompilerParams(use_tc_tiling_on_sc=False),
        scratch_types=[
            pltpu.VMEM((_NBLK, _B), jnp.int32),
            pltpu.VMEM((_NBLK, _B), jnp.int32),
            pltpu.VMEM((_NBUF, _B, width), jnp.float32),
            pltpu.VMEM_SHARED((_NPAD, width), jnp.float32),
            pltpu.SemaphoreType.DMA,
            pltpu.SemaphoreType.DMA,
            pltpu.SemaphoreType.DMA,
            pltpu.SemaphoreType.DMA,
        ],
    )


_deg_scatter = _make_edge_scatter(16, do_gather=False)
_g_scatter = _make_edge_scatter(64, do_gather=True)


def _mm_body(x_ref, w_ref, o_ref):
    o_ref[...] = jnp.dot(x_ref[...], w_ref[...], preferred_element_type=jnp.float32)


def _prep_body(degp_ref, h_ref, g_ref, dinv_ref):
    d = degp_ref[0] + degp_ref[1]
    deg = d[0:_N, 0:1] + 1.0
    dinv = lax.rsqrt(deg)
    dinv_ref[...] = dinv
    g_ref[...] = h_ref[...] * dinv


def _fin1_body(p_ref, g_ref, dinv_ref, b_ref, w_ref, o_ref):
    s = p_ref[0, 0:_N, :] + p_ref[1, 0:_N, :] + g_ref[...]
    h = jnp.tanh(s * dinv_ref[...] + b_ref[...])
    o_ref[...] = jnp.dot(h, w_ref[...], preferred_element_type=jnp.float32) * dinv_ref[...]


def _fin2_body(p_ref, g_ref, dinv_ref, b_ref, o_ref):
    s = p_ref[0, 0:_N, :] + p_ref[1, 0:_N, :] + g_ref[...]
    o_ref[...] = jnp.tanh(s * dinv_ref[...] + b_ref[...])


def _tc(body, *outs):
    return pl.pallas_call(body, out_shape=[jax.ShapeDtypeStruct(s, jnp.float32) for s in outs])


def kernel(x, edge_index, W1, b1, W2, b2):
    src = edge_index[0].astype(jnp.int32)
    dst = edge_index[1].astype(jnp.int32)
    e = src.shape[0]
    fill = _N + (jnp.arange(_EPAD - e, dtype=jnp.int32) % (_NPAD - _N))
    src_t = jnp.concatenate([src, fill]).reshape(_NT, _NBLK, _B)
    dst_t = jnp.concatenate([dst, fill]).reshape(_NT, _NBLK, _B)
    zeros16 = jnp.zeros((_NPAD, 16), jnp.float32)
    zeros64 = jnp.zeros((_NPAD, 64), jnp.float32)
    ones16 = jnp.ones((_B, 16), jnp.float32)

    degp = _deg_scatter(src_t, dst_t, ones16, zeros16)
    h1r, = _tc(_mm_body, (_N, 64))(x, W1)
    g1, dinv = _tc(_prep_body, (_N, 64), (_N, 1))(degp, h1r)
    g1p = jnp.pad(g1, ((0, _NPAD - _N), (0, 0)))
    p1 = _g_scatter(src_t, dst_t, g1p, zeros64)
    g2, = _tc(_fin1_body, (_N, 64))(p1, g1, dinv, b1.reshape(1, 64), W2)
    g2p = jnp.pad(g2, ((0, _NPAD - _N), (0, 0)))
    p2 = _g_scatter(src_t, dst_t, g2p, zeros64)
    out, = _tc(_fin2_body, (_N, 64))(p2, g2, dinv, b2.reshape(1, 64))
    return out


# speedup vs baseline: 42.2794x; 1.0282x over previous
import functools

import jax
import jax.numpy as jnp
from jax import lax
from jax.experimental import pallas as pl
from jax.experimental.pallas import tpu as pltpu
from jax.experimental.pallas import tpu_sc as plsc

_N = 10000
_NPAD = 10240
_B = 128
_NBLK = 80
_NT = 32
_EPAD = _NT * _NBLK * _B
_SLABC = 5
_NBUF = 4


def _make_edge_scatter(width, do_gather):
    mesh = plsc.VectorSubcoreMesh(core_axis_name="c", subcore_axis_name="s")

    def body(e_hbm, tab_hbm, zeros_hbm, out_hbm,
             src_v, dst_v, rows_v, acc_sh, sem0, sem1, sem2, sem3):
        sems = (sem0, sem1, sem2, sem3)
        cid = lax.axis_index("c")
        sid = lax.axis_index("s")
        t = cid * 16 + sid
        pltpu.sync_copy(e_hbm.at[1, t], dst_v)
        if do_gather:
            pltpu.sync_copy(e_hbm.at[0, t], src_v)
        else:
            pltpu.sync_copy(tab_hbm, rows_v.at[0])
        for c in range(_SLABC):
            rows = pl.ds((sid * _SLABC + c) * _B, _B)
            pltpu.sync_copy(zeros_hbm.at[rows], rows_v.at[1])
            pltpu.sync_copy(rows_v.at[1], acc_sh.at[rows])
        plsc.subcore_barrier()

        if do_gather:
            for b in range(_NBUF):
                pltpu.async_copy(tab_hbm.at[src_v.at[b]], rows_v.at[b], sems[b])

            def lbody(i, carry):
                j0 = i * _NBUF
                for b in range(_NBUF):
                    j = j0 + b
                    pltpu.make_async_copy(
                        tab_hbm.at[src_v.at[j]], rows_v.at[b], sems[b]).wait()
                    pltpu.sync_copy(rows_v.at[b], acc_sh.at[dst_v.at[j]], add=True)

                    @pl.when(j + _NBUF < _NBLK)
                    def _():
                        pltpu.async_copy(
                            tab_hbm.at[src_v.at[j + _NBUF]], rows_v.at[b], sems[b])
                return carry

            lax.fori_loop(0, _NBLK // _NBUF, lbody, 0)
        else:
            def lbody(j, carry):
                pltpu.sync_copy(rows_v.at[0], acc_sh.at[dst_v.at[j]], add=True)
                return carry

            lax.fori_loop(0, _NBLK, lbody, 0)

        plsc.subcore_barrier()
        for c in range(_SLABC):
            rows = pl.ds((sid * _SLABC + c) * _B, _B)
            pltpu.sync_copy(acc_sh.at[rows], out_hbm.at[cid, rows])

    return pl.kernel(
        body,
        out_type=jax.ShapeDtypeStruct((2, _NPAD, width), jnp.float32),
        mesh=mesh,
        compiler_params=pltpu.CompilerParams(use_tc_tiling_on_sc=False),
        scratch_types=[
            pltpu.VMEM((_NBLK, _B), jnp.int32),
            pltpu.VMEM((_NBLK, _B), jnp.int32),
            pltpu.VMEM((_NBUF, _B, width), jnp.float32),
            pltpu.VMEM_SHARED((_NPAD, width), jnp.float32),
            pltpu.SemaphoreType.DMA,
            pltpu.SemaphoreType.DMA,
            pltpu.SemaphoreType.DMA,
            pltpu.SemaphoreType.DMA,
        ],
    )


_deg_scatter = _make_edge_scatter(16, do_gather=False)
_g_scatter = _make_edge_scatter(64, do_gather=True)


def _mm_body(x_ref, w_ref, o_ref):
    o_ref[...] = jnp.dot(x_ref[...], w_ref[...], preferred_element_type=jnp.float32)


def _prep_body(degp_ref, h_ref, g_ref, dinv_ref):
    d = degp_ref[0] + degp_ref[1]
    deg = d[0:_N, 0:1] + 1.0
    dinv = lax.rsqrt(deg)
    dinv_ref[...] = dinv
    g_ref[0:_N, :] = h_ref[...] * dinv
    g_ref[_N:_NPAD, :] = jnp.zeros((_NPAD - _N, 64), jnp.float32)


def _fin1_body(p_ref, g_ref, dinv_ref, b_ref, w_ref, o_ref):
    s = p_ref[0, 0:_N, :] + p_ref[1, 0:_N, :] + g_ref[0:_N, :]
    h = jnp.tanh(s * dinv_ref[...] + b_ref[...])
    o_ref[0:_N, :] = jnp.dot(h, w_ref[...], preferred_element_type=jnp.float32) * dinv_ref[...]
    o_ref[_N:_NPAD, :] = jnp.zeros((_NPAD - _N, 64), jnp.float32)


def _fin2_body(p_ref, g_ref, dinv_ref, b_ref, o_ref):
    s = p_ref[0, 0:_N, :] + p_ref[1, 0:_N, :] + g_ref[0:_N, :]
    o_ref[...] = jnp.tanh(s * dinv_ref[...] + b_ref[...])


def _tc(body, *outs):
    return pl.pallas_call(body, out_shape=[jax.ShapeDtypeStruct(s, jnp.float32) for s in outs])


def kernel(x, edge_index, W1, b1, W2, b2):
    e = edge_index.shape[1]
    fill = _N + (jnp.arange(_EPAD - e, dtype=jnp.int32) % (_NPAD - _N))
    fill2 = jnp.broadcast_to(fill, (2, _EPAD - e))
    edges = jnp.concatenate(
        [edge_index.astype(jnp.int32), fill2], axis=1).reshape(2, _NT, _NBLK, _B)
    zeros16 = jnp.zeros((_NPAD, 16), jnp.float32)
    zeros64 = jnp.zeros((_NPAD, 64), jnp.float32)
    ones16 = jnp.ones((_B, 16), jnp.float32)

    degp = _deg_scatter(edges, ones16, zeros16)
    h1r, = _tc(_mm_body, (_N, 64))(x, W1)
    g1, dinv = _tc(_prep_body, (_NPAD, 64), (_N, 1))(degp, h1r)
    p1 = _g_scatter(edges, g1, zeros64)
    g2, = _tc(_fin1_body, (_NPAD, 64))(p1, g1, dinv, b1.reshape(1, 64), W2)
    p2 = _g_scatter(edges, g2, zeros64)
    out, = _tc(_fin2_body, (_N, 64))(p2, g2, dinv, b2.reshape(1, 64))
    return out


# speedup vs baseline: 42.5352x; 1.0060x over previous
import functools

import jax
import jax.numpy as jnp
from jax import lax
from jax.experimental import pallas as pl
from jax.experimental.pallas import tpu as pltpu
from jax.experimental.pallas import tpu_sc as plsc

_N = 10000
_NPAD = 10240
_B = 128
_NBLK = 80
_NT = 32
_EPAD = _NT * _NBLK * _B
_SLABC = 5
_NBUF = 8


def _make_edge_scatter(width, do_gather):
    mesh = plsc.VectorSubcoreMesh(core_axis_name="c", subcore_axis_name="s")

    def body(e_hbm, tab_hbm, zeros_hbm, out_hbm,
             src_v, dst_v, rows_v, acc_sh, gsem, ssem):
        cid = lax.axis_index("c")
        sid = lax.axis_index("s")
        t = cid * 16 + sid
        pltpu.async_copy(e_hbm.at[1, t], dst_v, gsem)
        if do_gather:
            pltpu.async_copy(e_hbm.at[0, t], src_v, gsem)
        else:
            pltpu.async_copy(tab_hbm, rows_v.at[7], gsem)
        for c in range(_SLABC):
            rows = pl.ds((sid * _SLABC + c) * _B, _B)
            pltpu.async_copy(zeros_hbm.at[rows], rows_v.at[c], gsem)
        pltpu.make_async_copy(e_hbm.at[1, t], dst_v, gsem).wait()
        if do_gather:
            pltpu.make_async_copy(e_hbm.at[0, t], src_v, gsem).wait()
        else:
            pltpu.make_async_copy(tab_hbm, rows_v.at[7], gsem).wait()
        for c in range(_SLABC):
            rows = pl.ds((sid * _SLABC + c) * _B, _B)
            pltpu.make_async_copy(zeros_hbm.at[rows], rows_v.at[c], gsem).wait()
            pltpu.async_copy(rows_v.at[c], acc_sh.at[rows], ssem)
        for c in range(_SLABC):
            rows = pl.ds((sid * _SLABC + c) * _B, _B)
            pltpu.make_async_copy(rows_v.at[c], acc_sh.at[rows], ssem).wait()
        plsc.subcore_barrier()

        if do_gather:
            ng = _NBLK // 4
            for b in range(4):
                pltpu.async_copy(tab_hbm.at[src_v.at[b]], rows_v.at[b], gsem)

            def gstep(i, carry):
                for half in range(2):
                    g = i * 2 + half
                    s0 = half * 4
                    o0 = 4 - half * 4
                    base = g * 4
                    for b in range(4):
                        pltpu.make_async_copy(
                            tab_hbm.at[src_v.at[base + b]],
                            rows_v.at[s0 + b], gsem).wait()
                    for b in range(4):
                        pltpu.async_copy(
                            rows_v.at[s0 + b], acc_sh.at[dst_v.at[base + b]],
                            ssem, add=True)

                    @pl.when(g > 0)
                    def _():
                        for b in range(4):
                            pltpu.make_async_copy(
                                rows_v.at[o0 + b],
                                acc_sh.at[dst_v.at[base - 4 + b]], ssem).wait()

                    @pl.when(g < ng - 1)
                    def _():
                        for b in range(4):
                            pltpu.async_copy(
                                tab_hbm.at[src_v.at[base + 4 + b]],
                                rows_v.at[o0 + b], gsem)
                return carry

            lax.fori_loop(0, ng // 2, gstep, 0)
            for b in range(4):
                pltpu.make_async_copy(
                    rows_v.at[4 + b],
                    acc_sh.at[dst_v.at[(ng - 1) * 4 + b]], ssem).wait()
        else:
            def lbody(i, carry):
                base = i * 8
                for b in range(8):
                    pltpu.async_copy(
                        rows_v.at[7], acc_sh.at[dst_v.at[base + b]],
                        ssem, add=True)

                @pl.when(i > 0)
                def _():
                    for b in range(8):
                        pltpu.make_async_copy(
                            rows_v.at[7], acc_sh.at[dst_v.at[base - 8 + b]],
                            ssem).wait()
                return carry

            lax.fori_loop(0, _NBLK // 8, lbody, 0)
            for b in range(8):
                pltpu.make_async_copy(
                    rows_v.at[7], acc_sh.at[dst_v.at[_NBLK - 8 + b]],
                    ssem).wait()

        plsc.subcore_barrier()
        for c in range(_SLABC):
            rows = pl.ds((sid * _SLABC + c) * _B, _B)
            pltpu.async_copy(acc_sh.at[rows], out_hbm.at[cid, rows], gsem)
        for c in range(_SLABC):
            rows = pl.ds((sid * _SLABC + c) * _B, _B)
            pltpu.make_async_copy(acc_sh.at[rows], out_hbm.at[cid, rows], gsem).wait()

    return pl.kernel(
        body,
        out_type=jax.ShapeDtypeStruct((2, _NPAD, width), jnp.float32),
        mesh=mesh,
        compiler_params=pltpu.CompilerParams(use_tc_tiling_on_sc=False),
        scratch_types=[
            pltpu.VMEM((_NBLK, _B), jnp.int32),
            pltpu.VMEM((_NBLK, _B), jnp.int32),
            pltpu.VMEM((_NBUF, _B, width), jnp.float32),
            pltpu.VMEM_SHARED((_NPAD, width), jnp.float32),
            pltpu.SemaphoreType.DMA,
            pltpu.SemaphoreType.DMA,
        ],
    )


_deg_scatter = _make_edge_scatter(16, do_gather=False)
_g_scatter = _make_edge_scatter(64, do_gather=True)


def _mm_body(x_ref, w_ref, o_ref):
    o_ref[...] = jnp.dot(x_ref[...], w_ref[...], preferred_element_type=jnp.float32)


def _prep_body(degp_ref, h_ref, g_ref, dinv_ref):
    d = degp_ref[0] + degp_ref[1]
    deg = d[0:_N, 0:1] + 1.0
    dinv = lax.rsqrt(deg)
    dinv_ref[...] = dinv
    g_ref[0:_N, :] = h_ref[...] * dinv
    g_ref[_N:_NPAD, :] = jnp.zeros((_NPAD - _N, 64), jnp.float32)


def _fin1_body(p_ref, g_ref, dinv_ref, b_ref, w_ref, o_ref):
    s = p_ref[0, 0:_N, :] + p_ref[1, 0:_N, :] + g_ref[0:_N, :]
    h = jnp.tanh(s * dinv_ref[...] + b_ref[...])
    o_ref[0:_N, :] = jnp.dot(h, w_ref[...], preferred_element_type=jnp.float32) * dinv_ref[...]
    o_ref[_N:_NPAD, :] = jnp.zeros((_NPAD - _N, 64), jnp.float32)


def _fin2_body(p_ref, g_ref, dinv_ref, b_ref, o_ref):
    s = p_ref[0, 0:_N, :] + p_ref[1, 0:_N, :] + g_ref[0:_N, :]
    o_ref[...] = jnp.tanh(s * dinv_ref[...] + b_ref[...])


def _tc(body, *outs):
    return pl.pallas_call(body, out_shape=[jax.ShapeDtypeStruct(s, jnp.float32) for s in outs])


def kernel(x, edge_index, W1, b1, W2, b2):
    e = edge_index.shape[1]
    fill = _N + (jnp.arange(_EPAD - e, dtype=jnp.int32) % (_NPAD - _N))
    fill2 = jnp.broadcast_to(fill, (2, _EPAD - e))
    edges = jnp.concatenate(
        [edge_index.astype(jnp.int32), fill2], axis=1).reshape(2, _NT, _NBLK, _B)
    zeros16 = jnp.zeros((_NPAD, 16), jnp.float32)
    zeros64 = jnp.zeros((_NPAD, 64), jnp.float32)
    ones16 = jnp.ones((_B, 16), jnp.float32)

    degp = _deg_scatter(edges, ones16, zeros16)
    h1r, = _tc(_mm_body, (_N, 64))(x, W1)
    g1, dinv = _tc(_prep_body, (_NPAD, 64), (_N, 1))(degp, h1r)
    p1 = _g_scatter(edges, g1, zeros64)
    g2, = _tc(_fin1_body, (_NPAD, 64))(p1, g1, dinv, b1.reshape(1, 64), W2)
    p2 = _g_scatter(edges, g2, zeros64)
    out, = _tc(_fin2_body, (_N, 64))(p2, g2, dinv, b2.reshape(1, 64))
    return out


# speedup vs baseline: 43.3368x; 1.0188x over previous
import functools

import jax
import jax.numpy as jnp
from jax import lax
from jax.experimental import pallas as pl
from jax.experimental.pallas import tpu as pltpu
from jax.experimental.pallas import tpu_sc as plsc

_N = 10000
_NPAD = 10240
_B = 128
_NBLK = 80
_NBLKT = 20
_NBLKS = 2500
_NT = 32
_SLABC = 5
_NBUF = 8


def _make_edge_scatter(width, do_gather):
    mesh = plsc.VectorSubcoreMesh(core_axis_name="c", subcore_axis_name="s")

    def body(e_hbm, tab_hbm, zeros_hbm, out_hbm,
             src_v, dst_v, rows_v, acc_sh, gsem, ssem):
        cid = lax.axis_index("c")
        sid = lax.axis_index("s")
        t = cid * 16 + sid
        for c in range(_SLABC):
            rows = pl.ds((sid * _SLABC + c) * _B, _B)
            pltpu.async_copy(zeros_hbm.at[rows], acc_sh.at[rows], gsem)

        @pl.when(t < _NT - 1)
        def _():
            blks = pl.ds(t * _NBLK, _NBLK)
            pltpu.async_copy(e_hbm.at[1, blks], dst_v, gsem)
            if do_gather:
                pltpu.async_copy(e_hbm.at[0, blks], src_v, gsem)
            pltpu.make_async_copy(e_hbm.at[1, blks], dst_v, gsem).wait()
            if do_gather:
                pltpu.make_async_copy(e_hbm.at[0, blks], src_v, gsem).wait()

        @pl.when(t == _NT - 1)
        def _():
            blks = pl.ds((_NT - 1) * _NBLK, _NBLKT)
            dslab = dst_v.at[pl.ds(0, _NBLKT)]
            pltpu.async_copy(e_hbm.at[1, blks], dslab, gsem)
            if do_gather:
                sslab = src_v.at[pl.ds(0, _NBLKT)]
                pltpu.async_copy(e_hbm.at[0, blks], sslab, gsem)
                pltpu.make_async_copy(e_hbm.at[0, blks], sslab, gsem).wait()
            pltpu.make_async_copy(e_hbm.at[1, blks], dslab, gsem).wait()

        if not do_gather:
            pltpu.async_copy(tab_hbm, rows_v.at[7], gsem)
            pltpu.make_async_copy(tab_hbm, rows_v.at[7], gsem).wait()
        for c in range(_SLABC):
            rows = pl.ds((sid * _SLABC + c) * _B, _B)
            pltpu.make_async_copy(zeros_hbm.at[rows], acc_sh.at[rows], gsem).wait()
        plsc.subcore_barrier()

        if do_gather:
            def run_pipe(nblk):
                ng = nblk // 4
                for b in range(4):
                    pltpu.async_copy(tab_hbm.at[src_v.at[b]], rows_v.at[b], gsem)

                def gstep(i, carry):
                    for half in range(2):
                        g = i * 2 + half
                        s0 = half * 4
                        o0 = 4 - half * 4
                        base = g * 4
                        for b in range(4):
                            pltpu.make_async_copy(
                                tab_hbm.at[src_v.at[base + b]],
                                rows_v.at[s0 + b], gsem).wait()
                        for b in range(4):
                            pltpu.async_copy(
                                rows_v.at[s0 + b], acc_sh.at[dst_v.at[base + b]],
                                ssem, add=True)

                        @pl.when(g > 0)
                        def _():
                            for b in range(4):
                                pltpu.make_async_copy(
                                    rows_v.at[o0 + b],
                                    acc_sh.at[dst_v.at[base - 4 + b]],
                                    ssem).wait()

                        @pl.when(g < ng - 1)
                        def _():
                            for b in range(4):
                                pltpu.async_copy(
                                    tab_hbm.at[src_v.at[base + 4 + b]],
                                    rows_v.at[o0 + b], gsem)
                    return carry

                lax.fori_loop(0, ng // 2, gstep, 0)
                for b in range(4):
                    pltpu.make_async_copy(
                        rows_v.at[4 + b],
                        acc_sh.at[dst_v.at[(ng - 1) * 4 + b]], ssem).wait()

            @pl.when(t < _NT - 1)
            def _():
                run_pipe(_NBLK)

            @pl.when(t == _NT - 1)
            def _():
                run_pipe(_NBLKT - 4)
                base = _NBLKT - 4
                for b in range(4):
                    pltpu.async_copy(
                        tab_hbm.at[src_v.at[base + b]], rows_v.at[b], gsem)
                for b in range(4):
                    pltpu.make_async_copy(
                        tab_hbm.at[src_v.at[base + b]], rows_v.at[b], gsem).wait()
                    pltpu.async_copy(
                        rows_v.at[b], acc_sh.at[dst_v.at[base + b]],
                        ssem, add=True)
                for b in range(4):
                    pltpu.make_async_copy(
                        rows_v.at[b], acc_sh.at[dst_v.at[base + b]], ssem).wait()
        else:
            def run_hist(nblk):
                def lbody(i, carry):
                    base = i * 8
                    for b in range(8):
                        pltpu.async_copy(
                            rows_v.at[7], acc_sh.at[dst_v.at[base + b]],
                            ssem, add=True)

                    @pl.when(i > 0)
                    def _():
                        for b in range(8):
                            pltpu.make_async_copy(
                                rows_v.at[7], acc_sh.at[dst_v.at[base - 8 + b]],
                                ssem).wait()
                    return carry

                lax.fori_loop(0, nblk // 8, lbody, 0)
                for b in range(8):
                    pltpu.make_async_copy(
                        rows_v.at[7], acc_sh.at[dst_v.at[nblk - 8 + b]],
                        ssem).wait()

            @pl.when(t < _NT - 1)
            def _():
                run_hist(_NBLK)

            @pl.when(t == _NT - 1)
            def _():
                for b in range(8):
                    pltpu.async_copy(
                        rows_v.at[7], acc_sh.at[dst_v.at[b]], ssem, add=True)
                for g in range(3):
                    for b in range(4):
                        pltpu.make_async_copy(
                            rows_v.at[7], acc_sh.at[dst_v.at[g * 4 + b]],
                            ssem).wait()
                    for b in range(4):
                        pltpu.async_copy(
                            rows_v.at[7], acc_sh.at[dst_v.at[8 + g * 4 + b]],
                            ssem, add=True)
                for b in range(8):
                    pltpu.make_async_copy(
                        rows_v.at[7], acc_sh.at[dst_v.at[12 + b]], ssem).wait()

        plsc.subcore_barrier()
        for c in range(_SLABC):
            rows = pl.ds((sid * _SLABC + c) * _B, _B)
            pltpu.async_copy(acc_sh.at[rows], out_hbm.at[cid, rows], gsem)
        for c in range(_SLABC):
            rows = pl.ds((sid * _SLABC + c) * _B, _B)
            pltpu.make_async_copy(acc_sh.at[rows], out_hbm.at[cid, rows], gsem).wait()

    return pl.kernel(
        body,
        out_type=jax.ShapeDtypeStruct((2, _NPAD, width), jnp.float32),
        mesh=mesh,
        compiler_params=pltpu.CompilerParams(use_tc_tiling_on_sc=False),
        scratch_types=[
            pltpu.VMEM((_NBLK, _B), jnp.int32),
            pltpu.VMEM((_NBLK, _B), jnp.int32),
            pltpu.VMEM((_NBUF, _B, width), jnp.float32),
            pltpu.VMEM_SHARED((_NPAD, width), jnp.float32),
            pltpu.SemaphoreType.DMA,
            pltpu.SemaphoreType.DMA,
        ],
    )


_deg_scatter = _make_edge_scatter(16, do_gather=False)
_g_scatter = _make_edge_scatter(64, do_gather=True)


def _mm_body(x_ref, w_ref, o_ref):
    o_ref[...] = jnp.dot(x_ref[...], w_ref[...], preferred_element_type=jnp.float32)


def _prep_body(degp_ref, h_ref, g_ref, dinv_ref):
    d = degp_ref[0] + degp_ref[1]
    deg = d[0:_N, 0:1] + 1.0
    dinv = lax.rsqrt(deg)
    dinv_ref[...] = dinv
    g_ref[0:_N, :] = h_ref[...] * dinv
    g_ref[_N:_NPAD, :] = jnp.zeros((_NPAD - _N, 64), jnp.float32)


def _fin1_body(p_ref, g_ref, dinv_ref, b_ref, w_ref, o_ref):
    s = p_ref[0, 0:_N, :] + p_ref[1, 0:_N, :] + g_ref[0:_N, :]
    h = jnp.tanh(s * dinv_ref[...] + b_ref[...])
    o_ref[0:_N, :] = jnp.dot(h, w_ref[...], preferred_element_type=jnp.float32) * dinv_ref[...]
    o_ref[_N:_NPAD, :] = jnp.zeros((_NPAD - _N, 64), jnp.float32)


def _fin2_body(p_ref, g_ref, dinv_ref, b_ref, o_ref):
    s = p_ref[0, 0:_N, :] + p_ref[1, 0:_N, :] + g_ref[0:_N, :]
    o_ref[...] = jnp.tanh(s * dinv_ref[...] + b_ref[...])


def _tc(body, *outs):
    return pl.pallas_call(body, out_shape=[jax.ShapeDtypeStruct(s, jnp.float32) for s in outs])


def kernel(x, edge_index, W1, b1, W2, b2):
    edges = edge_index.astype(jnp.int32).reshape(2, _NBLKS, _B)
    zeros16 = jnp.zeros((_NPAD, 16), jnp.float32)
    zeros64 = jnp.zeros((_NPAD, 64), jnp.float32)
    ones16 = jnp.ones((_B, 16), jnp.float32)

    degp = _deg_scatter(edges, ones16, zeros16)
    h1r, = _tc(_mm_body, (_N, 64))(x, W1)
    g1, dinv = _tc(_prep_body, (_NPAD, 64), (_N, 1))(degp, h1r)
    p1 = _g_scatter(edges, g1, zeros64)
    g2, = _tc(_fin1_body, (_NPAD, 64))(p1, g1, dinv, b1.reshape(1, 64), W2)
    p2 = _g_scatter(edges, g2, zeros64)
    out, = _tc(_fin2_body, (_N, 64))(p2, g2, dinv, b2.reshape(1, 64))
    return out


# speedup vs baseline: 43.5693x; 1.0054x over previous
import functools

import jax
import jax.numpy as jnp
from jax import lax
from jax.experimental import pallas as pl
from jax.experimental.pallas import tpu as pltpu
from jax.experimental.pallas import tpu_sc as plsc

_N = 10000
_NPAD = 10240
_B = 128
_NBLK = 80
_NBLKT = 20
_NBLKS = 2500
_NT = 32
_SLABC = 5
_NBUF = 8


def _make_edge_scatter(width, do_gather):
    mesh = plsc.VectorSubcoreMesh(core_axis_name="c", subcore_axis_name="s")

    def body(e_hbm, tab_hbm, zeros_hbm, out_hbm,
             src_v, dst_v, rows_v, acc_sh, gsem, ssem):
        cid = lax.axis_index("c")
        sid = lax.axis_index("s")
        t = cid * 16 + sid
        for c in range(_SLABC):
            rows = pl.ds((sid * _SLABC + c) * _B, _B)
            pltpu.async_copy(zeros_hbm.at[rows], acc_sh.at[rows], gsem)

        @pl.when(t < _NT - 1)
        def _():
            blks = pl.ds(t * _NBLK, _NBLK)
            pltpu.async_copy(e_hbm.at[1, blks], dst_v, gsem)
            if do_gather:
                pltpu.async_copy(e_hbm.at[0, blks], src_v, gsem)
            pltpu.make_async_copy(e_hbm.at[1, blks], dst_v, gsem).wait()
            if do_gather:
                pltpu.make_async_copy(e_hbm.at[0, blks], src_v, gsem).wait()

        @pl.when(t == _NT - 1)
        def _():
            blks = pl.ds((_NT - 1) * _NBLK, _NBLKT)
            dslab = dst_v.at[pl.ds(0, _NBLKT)]
            pltpu.async_copy(e_hbm.at[1, blks], dslab, gsem)
            if do_gather:
                sslab = src_v.at[pl.ds(0, _NBLKT)]
                pltpu.async_copy(e_hbm.at[0, blks], sslab, gsem)
                pltpu.make_async_copy(e_hbm.at[0, blks], sslab, gsem).wait()
            pltpu.make_async_copy(e_hbm.at[1, blks], dslab, gsem).wait()

        if not do_gather:
            pltpu.async_copy(tab_hbm, rows_v.at[7], gsem)
            pltpu.make_async_copy(tab_hbm, rows_v.at[7], gsem).wait()
        for c in range(_SLABC):
            rows = pl.ds((sid * _SLABC + c) * _B, _B)
            pltpu.make_async_copy(zeros_hbm.at[rows], acc_sh.at[rows], gsem).wait()
        plsc.subcore_barrier()

        if do_gather:
            def run_pipe(nblk):
                ng = nblk // 4
                for b in range(4):
                    pltpu.async_copy(tab_hbm.at[src_v.at[b]], rows_v.at[b], gsem)

                def gstep(i, carry):
                    for half in range(2):
                        g = i * 2 + half
                        s0 = half * 4
                        o0 = 4 - half * 4
                        base = g * 4
                        for b in range(4):
                            pltpu.make_async_copy(
                                tab_hbm.at[src_v.at[base + b]],
                                rows_v.at[s0 + b], gsem).wait()
                        for b in range(4):
                            pltpu.async_copy(
                                rows_v.at[s0 + b], acc_sh.at[dst_v.at[base + b]],
                                ssem, add=True)

                        @pl.when(g > 0)
                        def _():
                            for b in range(4):
                                pltpu.make_async_copy(
                                    rows_v.at[o0 + b],
                                    acc_sh.at[dst_v.at[base - 4 + b]],
                                    ssem).wait()

                        @pl.when(g < ng - 1)
                        def _():
                            for b in range(4):
                                pltpu.async_copy(
                                    tab_hbm.at[src_v.at[base + 4 + b]],
                                    rows_v.at[o0 + b], gsem)
                    return carry

                lax.fori_loop(0, ng // 2, gstep, 0)
                for b in range(4):
                    pltpu.make_async_copy(
                        rows_v.at[4 + b],
                        acc_sh.at[dst_v.at[(ng - 1) * 4 + b]], ssem).wait()

            @pl.when(t < _NT - 1)
            def _():
                run_pipe(_NBLK)

            @pl.when(t == _NT - 1)
            def _():
                run_pipe(_NBLKT - 4)
                base = _NBLKT - 4
                for b in range(4):
                    pltpu.async_copy(
                        tab_hbm.at[src_v.at[base + b]], rows_v.at[b], gsem)
                for b in range(4):
                    pltpu.make_async_copy(
                        tab_hbm.at[src_v.at[base + b]], rows_v.at[b], gsem).wait()
                    pltpu.async_copy(
                        rows_v.at[b], acc_sh.at[dst_v.at[base + b]],
                        ssem, add=True)
                for b in range(4):
                    pltpu.make_async_copy(
                        rows_v.at[b], acc_sh.at[dst_v.at[base + b]], ssem).wait()
        else:
            def run_hist(nblk):
                def lbody(i, carry):
                    base = i * 8
                    for b in range(8):
                        pltpu.async_copy(
                            rows_v.at[7], acc_sh.at[dst_v.at[base + b]],
                            ssem, add=True)

                    @pl.when(i > 0)
                    def _():
                        for b in range(8):
                            pltpu.make_async_copy(
                                rows_v.at[7], acc_sh.at[dst_v.at[base - 8 + b]],
                                ssem).wait()
                    return carry

                lax.fori_loop(0, nblk // 8, lbody, 0)
                for b in range(8):
                    pltpu.make_async_copy(
                        rows_v.at[7], acc_sh.at[dst_v.at[nblk - 8 + b]],
                        ssem).wait()

            @pl.when(t < _NT - 1)
            def _():
                run_hist(_NBLK)

            @pl.when(t == _NT - 1)
            def _():
                for b in range(8):
                    pltpu.async_copy(
                        rows_v.at[7], acc_sh.at[dst_v.at[b]], ssem, add=True)
                for g in range(3):
                    for b in range(4):
                        pltpu.make_async_copy(
                            rows_v.at[7], acc_sh.at[dst_v.at[g * 4 + b]],
                            ssem).wait()
                    for b in range(4):
                        pltpu.async_copy(
                            rows_v.at[7], acc_sh.at[dst_v.at[8 + g * 4 + b]],
                            ssem, add=True)
                for b in range(8):
                    pltpu.make_async_copy(
                        rows_v.at[7], acc_sh.at[dst_v.at[12 + b]], ssem).wait()

        plsc.subcore_barrier()
        for c in range(_SLABC):
            rows = pl.ds((sid * _SLABC + c) * _B, _B)
            pltpu.async_copy(acc_sh.at[rows], out_hbm.at[cid, rows], gsem)
        for c in range(_SLABC):
            rows = pl.ds((sid * _SLABC + c) * _B, _B)
            pltpu.make_async_copy(acc_sh.at[rows], out_hbm.at[cid, rows], gsem).wait()

    return pl.kernel(
        body,
        out_type=jax.ShapeDtypeStruct((2, _NPAD, width), jnp.float32),
        mesh=mesh,
        compiler_params=pltpu.CompilerParams(use_tc_tiling_on_sc=False),
        scratch_types=[
            pltpu.VMEM((_NBLK, _B), jnp.int32),
            pltpu.VMEM((_NBLK, _B), jnp.int32),
            pltpu.VMEM((_NBUF, _B, width), jnp.float32),
            pltpu.VMEM_SHARED((_NPAD, width), jnp.float32),
            pltpu.SemaphoreType.DMA,
            pltpu.SemaphoreType.DMA,
        ],
    )


_deg_scatter = _make_edge_scatter(16, do_gather=False)
_g_scatter = _make_edge_scatter(64, do_gather=True)


def _prep_body(degp_ref, x_ref, w_ref, g_ref, dinv_ref):
    d = degp_ref[0] + degp_ref[1]
    deg = d[0:_N, 0:1] + 1.0
    dinv = lax.rsqrt(deg)
    dinv_ref[...] = dinv
    h = jnp.dot(x_ref[...], w_ref[...], preferred_element_type=jnp.float32)
    g_ref[0:_N, :] = h * dinv
    g_ref[_N:_NPAD, :] = jnp.zeros((_NPAD - _N, 64), jnp.float32)


def _fin1_body(p_ref, g_ref, dinv_ref, b_ref, w_ref, o_ref):
    s = p_ref[0, 0:_N, :] + p_ref[1, 0:_N, :] + g_ref[0:_N, :]
    h = jnp.tanh(s * dinv_ref[...] + b_ref[...])
    o_ref[0:_N, :] = jnp.dot(h, w_ref[...], preferred_element_type=jnp.float32) * dinv_ref[...]
    o_ref[_N:_NPAD, :] = jnp.zeros((_NPAD - _N, 64), jnp.float32)


def _fin2_body(p_ref, g_ref, dinv_ref, b_ref, o_ref):
    s = p_ref[0, 0:_N, :] + p_ref[1, 0:_N, :] + g_ref[0:_N, :]
    o_ref[...] = jnp.tanh(s * dinv_ref[...] + b_ref[...])


def _tc(body, *outs):
    return pl.pallas_call(body, out_shape=[jax.ShapeDtypeStruct(s, jnp.float32) for s in outs])


def kernel(x, edge_index, W1, b1, W2, b2):
    edges = edge_index.astype(jnp.int32).reshape(2, _NBLKS, _B)
    zeros16 = jnp.zeros((_NPAD, 16), jnp.float32)
    zeros64 = jnp.zeros((_NPAD, 64), jnp.float32)
    ones16 = jnp.ones((_B, 16), jnp.float32)

    degp = _deg_scatter(edges, ones16, zeros16)
    g1, dinv = _tc(_prep_body, (_NPAD, 64), (_N, 1))(degp, x, W1)
    p1 = _g_scatter(edges, g1, zeros64)
    g2, = _tc(_fin1_body, (_NPAD, 64))(p1, g1, dinv, b1.reshape(1, 64), W2)
    p2 = _g_scatter(edges, g2, zeros64)
    out, = _tc(_fin2_body, (_N, 64))(p2, g2, dinv, b2.reshape(1, 64))
    return out
